# Initial kernel scaffold; baseline (speedup 1.0000x reference)
import jax
import jax.numpy as jnp
from jax.experimental import pallas as pl


def kernel(inp, norm_g, qkv_w, mem_kv, k_pos, v_pos, k_cw, k_cb, v_cw, v_cb, strat_w, strat_b, out_w):
    raise NotImplementedError("write your pallas kernel here")


# speedup vs baseline: 6.5335x; 6.5335x over previous
import functools

import jax
import jax.numpy as jnp
import numpy as np
from jax.experimental import pallas as pl
from jax.experimental.pallas import tpu as pltpu

B = 1
N = 2048
DIM = 768
HEADS = 12
DIM_HEAD = 32
DIM_INNER = HEADS * DIM_HEAD
BLOCK = 16
NUM_SEL = 4
NUM_MEM = 4
WINDOW = 64
NC = N // BLOCK

EPS = 1.1920929e-07
MASKVAL = -jnp.finfo(jnp.float32).max
SCALE = DIM_HEAD ** -0.5

QB = 256
KB = 256


def _proj_kernel(x_ref, g_ref, wq_ref, ws_ref, bs_ref, qkv_ref, comb_ref):
    x = x_ref[...]
    ms = jnp.mean(x * x, axis=1, keepdims=True)
    xn = x * jax.lax.rsqrt(ms + EPS) * g_ref[...]
    qkv_ref[...] = jnp.dot(xn, wq_ref[...], preferred_element_type=jnp.float32)
    comb_ref[...] = jax.nn.sigmoid(
        jnp.dot(xn, ws_ref[...], preferred_element_type=jnp.float32) + bs_ref[...])


def _comp_kernel(q_ref, kb_ref, vb_ref, kw_ref, vw_ref, kcb_ref, vcb_ref,
                 kpos_ref, vpos_ref, memk_ref, memv_ref,
                 comp_ref, selidx_ref, selval_ref):
    q = q_ref[0]
    ck = jnp.dot(kb_ref[0] + kpos_ref[0], kw_ref[0],
                 preferred_element_type=jnp.float32) + kcb_ref[0]
    cv = jnp.dot(vb_ref[0] + vpos_ref[0], vw_ref[0],
                 preferred_element_type=jnp.float32) + vcb_ref[0]

    dn = (((1,), (1,)), ((), ()))
    sim_mem = jax.lax.dot_general(q, memk_ref[0], dn,
                                  preferred_element_type=jnp.float32) * SCALE
    sim_ck = jax.lax.dot_general(q, ck, dn,
                                 preferred_element_type=jnp.float32) * SCALE

    ii = jax.lax.broadcasted_iota(jnp.int32, (N, NC), 0)
    cc = jax.lax.broadcasted_iota(jnp.int32, (N, NC), 1)
    vis = cc < (ii // BLOCK)
    sim_ck = jnp.where(vis, sim_ck, MASKVAL)

    m = jnp.maximum(jnp.max(sim_mem, axis=1, keepdims=True),
                    jnp.max(sim_ck, axis=1, keepdims=True))
    e_mem = jnp.exp(sim_mem - m)
    e_ck = jnp.exp(sim_ck - m)
    l = (jnp.sum(e_mem, axis=1, keepdims=True)
         + jnp.sum(e_ck, axis=1, keepdims=True))
    comp_ref[0] = (jnp.dot(e_mem, memv_ref[0], preferred_element_type=jnp.float32)
                   + jnp.dot(e_ck, cv, preferred_element_type=jnp.float32)) / l

    val = e_ck / l
    work = val
    idx_cols = []
    val_cols = []
    for _ in range(NUM_SEL):
        mx = jnp.max(work, axis=1, keepdims=True)
        idx = jnp.min(jnp.where(work == mx, cc, NC), axis=1, keepdims=True)
        idx_cols.append(idx)
        val_cols.append(mx)
        work = jnp.where(cc == idx, -1.0, work)
    selidx_ref[0] = jnp.concatenate(idx_cols, axis=1)
    selval_ref[0] = jnp.concatenate(val_cols, axis=1)


def _attn_kernel(q_ref, k_ref, v_ref, cosq_ref, sinq_ref, cosk_ref, sink_ref,
                 p_ref, sel_ref, sval_ref, comp_ref, gates_ref, out_ref):
    qb = pl.program_id(1)
    pmat = p_ref[...]

    q = q_ref[0]
    rq = (q * cosq_ref[...]
          + jnp.dot(q, pmat, preferred_element_type=jnp.float32)
          * sinq_ref[...]) * SCALE

    sel = sel_ref[0]
    sval = sval_ref[0]

    ri = jax.lax.broadcasted_iota(jnp.int32, (QB, KB), 0)
    rj = jax.lax.broadcasted_iota(jnp.int32, (QB, KB), 1)
    gi = qb * QB + ri

    def body(kb, carry):
        mf, lf, accf, mw, lw, accw = carry
        ks = k_ref[0, pl.ds(kb * KB, KB), :]
        vs = v_ref[0, pl.ds(kb * KB, KB), :]
        ck = cosk_ref[pl.ds(kb * KB, KB), :]
        sk = sink_ref[pl.ds(kb * KB, KB), :]
        rk = ks * ck + jnp.dot(ks, pmat, preferred_element_type=jnp.float32) * sk

        s = jax.lax.dot_general(rq, rk, (((1,), (1,)), ((), ())),
                                preferred_element_type=jnp.float32)

        gj = kb * KB + rj
        causal = gj <= gi
        swm = causal & (gi - gj <= WINDOW)
        own = causal & ((gi // BLOCK) == (gj // BLOCK))
        jblk = gj // BLOCK
        fm = own
        for t in range(NUM_SEL):
            fm = fm | ((sel[:, t:t + 1] == jblk) & (sval[:, t:t + 1] > 1e-10))

        sf = jnp.where(fm, s, MASKVAL)
        sw = jnp.where(swm, s, MASKVAL)

        mf_n = jnp.maximum(mf, jnp.max(sf, axis=1, keepdims=True))
        pf = jnp.where(fm, jnp.exp(sf - mf_n), 0.0)
        cf = jnp.exp(mf - mf_n)
        lf = lf * cf + jnp.sum(pf, axis=1, keepdims=True)
        accf = accf * cf + jnp.dot(pf, vs, preferred_element_type=jnp.float32)

        mw_n = jnp.maximum(mw, jnp.max(sw, axis=1, keepdims=True))
        pw = jnp.where(swm, jnp.exp(sw - mw_n), 0.0)
        cw = jnp.exp(mw - mw_n)
        lw = lw * cw + jnp.sum(pw, axis=1, keepdims=True)
        accw = accw * cw + jnp.dot(pw, vs, preferred_element_type=jnp.float32)

        return mf_n, lf, accf, mw_n, lw, accw

    init = (jnp.full((QB, 1), MASKVAL), jnp.zeros((QB, 1)),
            jnp.zeros((QB, DIM_HEAD)),
            jnp.full((QB, 1), MASKVAL), jnp.zeros((QB, 1)),
            jnp.zeros((QB, DIM_HEAD)))
    mf, lf, accf, mw, lw, accw = jax.lax.fori_loop(0, qb + 1, body, init)

    fine = accf / lf
    swout = accw / lw
    g = gates_ref[0]
    out_ref[0] = (g[:, 0:1] * comp_ref[0] + g[:, 1:2] * fine
                  + g[:, 2:3] * swout)


def _out_kernel(x_ref, w_ref, o_ref):
    o_ref[...] = jnp.dot(x_ref[...], w_ref[...],
                         preferred_element_type=jnp.float32)


def _rotary_tables():
    inv = 1.0 / (10000.0 ** (np.arange(0, DIM_HEAD, 2, dtype=np.float32)
                             / DIM_HEAD))
    freqs = np.arange(N, dtype=np.float32)[:, None] * inv[None, :]
    freqs = np.repeat(freqs, 2, axis=-1)
    cos = jnp.asarray(np.cos(freqs))
    sin = jnp.asarray(np.sin(freqs))
    p = np.zeros((DIM_HEAD, DIM_HEAD), np.float32)
    for j in range(0, DIM_HEAD, 2):
        p[j + 1, j] = -1.0
        p[j, j + 1] = 1.0
    return cos, sin, jnp.asarray(p)


def kernel(inp, norm_g, qkv_w, mem_kv, k_pos, v_pos, k_cw, k_cb, v_cw, v_cb,
           strat_w, strat_b, out_w):
    x2 = inp.reshape(N, DIM)
    gate_pad = 128
    ws = jnp.zeros((DIM, gate_pad), jnp.float32).at[:, :3 * HEADS].set(strat_w)
    bs = jnp.zeros((1, gate_pad), jnp.float32).at[0, :3 * HEADS].set(strat_b)

    qkv, comb = pl.pallas_call(
        _proj_kernel,
        grid=(N // QB,),
        in_specs=[
            pl.BlockSpec((QB, DIM), lambda i: (i, 0)),
            pl.BlockSpec((1, DIM), lambda i: (0, 0)),
            pl.BlockSpec((DIM, 3 * DIM_INNER), lambda i: (0, 0)),
            pl.BlockSpec((DIM, gate_pad), lambda i: (0, 0)),
            pl.BlockSpec((1, gate_pad), lambda i: (0, 0)),
        ],
        out_specs=[
            pl.BlockSpec((QB, 3 * DIM_INNER), lambda i: (i, 0)),
            pl.BlockSpec((QB, gate_pad), lambda i: (i, 0)),
        ],
        out_shape=[
            jax.ShapeDtypeStruct((N, 3 * DIM_INNER), jnp.float32),
            jax.ShapeDtypeStruct((N, gate_pad), jnp.float32),
        ],
    )(x2, norm_g.reshape(1, DIM), qkv_w, ws, bs)

    q = qkv[:, :DIM_INNER].reshape(N, HEADS, DIM_HEAD).transpose(1, 0, 2)
    k = qkv[:, DIM_INNER:2 * DIM_INNER].reshape(N, HEADS, DIM_HEAD).transpose(1, 0, 2)
    v = qkv[:, 2 * DIM_INNER:].reshape(N, HEADS, DIM_HEAD).transpose(1, 0, 2)
    gates = comb[:, :3 * HEADS].reshape(N, HEADS, 3).transpose(1, 0, 2)

    kblk = k.reshape(HEADS, NC, BLOCK * DIM_HEAD)
    vblk = v.reshape(HEADS, NC, BLOCK * DIM_HEAD)
    kw = k_cw.transpose(0, 3, 2, 1).reshape(HEADS, BLOCK * DIM_HEAD, DIM_HEAD)
    vw = v_cw.transpose(0, 3, 2, 1).reshape(HEADS, BLOCK * DIM_HEAD, DIM_HEAD)
    kpos_f = k_pos.reshape(HEADS, 1, BLOCK * DIM_HEAD)
    vpos_f = v_pos.reshape(HEADS, 1, BLOCK * DIM_HEAD)
    kcb2 = k_cb.reshape(HEADS, 1, DIM_HEAD)
    vcb2 = v_cb.reshape(HEADS, 1, DIM_HEAD)
    mem_k = mem_kv[0]
    mem_v = mem_kv[1]

    hfix = lambda shape: pl.BlockSpec((1,) + shape, lambda h: (h, 0, 0))
    comp, sel_idx, sel_val = pl.pallas_call(
        _comp_kernel,
        grid=(HEADS,),
        in_specs=[
            hfix((N, DIM_HEAD)),
            hfix((NC, BLOCK * DIM_HEAD)),
            hfix((NC, BLOCK * DIM_HEAD)),
            hfix((BLOCK * DIM_HEAD, DIM_HEAD)),
            hfix((BLOCK * DIM_HEAD, DIM_HEAD)),
            hfix((1, DIM_HEAD)),
            hfix((1, DIM_HEAD)),
            hfix((1, BLOCK * DIM_HEAD)),
            hfix((1, BLOCK * DIM_HEAD)),
            hfix((NUM_MEM, DIM_HEAD)),
            hfix((NUM_MEM, DIM_HEAD)),
        ],
        out_specs=[
            hfix((N, DIM_HEAD)),
            hfix((N, NUM_SEL)),
            hfix((N, NUM_SEL)),
        ],
        out_shape=[
            jax.ShapeDtypeStruct((HEADS, N, DIM_HEAD), jnp.float32),
            jax.ShapeDtypeStruct((HEADS, N, NUM_SEL), jnp.int32),
            jax.ShapeDtypeStruct((HEADS, N, NUM_SEL), jnp.float32),
        ],
    )(q, kblk, vblk, kw, vw, kcb2, vcb2, kpos_f, vpos_f, mem_k, mem_v)

    cos, sin, pmat = _rotary_tables()

    outc = pl.pallas_call(
        _attn_kernel,
        grid=(HEADS, N // QB),
        in_specs=[
            pl.BlockSpec((1, QB, DIM_HEAD), lambda h, i: (h, i, 0)),
            pl.BlockSpec((1, N, DIM_HEAD), lambda h, i: (h, 0, 0)),
            pl.BlockSpec((1, N, DIM_HEAD), lambda h, i: (h, 0, 0)),
            pl.BlockSpec((QB, DIM_HEAD), lambda h, i: (i, 0)),
            pl.BlockSpec((QB, DIM_HEAD), lambda h, i: (i, 0)),
            pl.BlockSpec((N, DIM_HEAD), lambda h, i: (0, 0)),
            pl.BlockSpec((N, DIM_HEAD), lambda h, i: (0, 0)),
            pl.BlockSpec((DIM_HEAD, DIM_HEAD), lambda h, i: (0, 0)),
            pl.BlockSpec((1, QB, NUM_SEL), lambda h, i: (h, i, 0)),
            pl.BlockSpec((1, QB, NUM_SEL), lambda h, i: (h, i, 0)),
            pl.BlockSpec((1, QB, DIM_HEAD), lambda h, i: (h, i, 0)),
            pl.BlockSpec((1, QB, 3), lambda h, i: (h, i, 0)),
        ],
        out_specs=pl.BlockSpec((1, QB, DIM_HEAD), lambda h, i: (h, i, 0)),
        out_shape=jax.ShapeDtypeStruct((HEADS, N, DIM_HEAD), jnp.float32),
    )(q, k, v, cos, sin, cos, sin, pmat, sel_idx, sel_val, comp, gates)

    merged = outc.transpose(1, 0, 2).reshape(N, DIM_INNER)

    out = pl.pallas_call(
        _out_kernel,
        grid=(N // QB,),
        in_specs=[
            pl.BlockSpec((QB, DIM_INNER), lambda i: (i, 0)),
            pl.BlockSpec((DIM_INNER, DIM), lambda i: (0, 0)),
        ],
        out_specs=pl.BlockSpec((QB, DIM), lambda i: (i, 0)),
        out_shape=jax.ShapeDtypeStruct((N, DIM), jnp.float32),
    )(merged, out_w)

    return out.reshape(B, N, DIM)


# speedup vs baseline: 11.2065x; 1.7152x over previous
import functools

import jax
import jax.numpy as jnp
import numpy as np
from jax.experimental import pallas as pl
from jax.experimental.pallas import tpu as pltpu

B = 1
N = 2048
DIM = 768
HEADS = 12
DIM_HEAD = 32
DIM_INNER = HEADS * DIM_HEAD
BLOCK = 16
NUM_SEL = 4
NUM_MEM = 4
WINDOW = 64
NC = N // BLOCK

EPS = 1.1920929e-07
MASKVAL = -jnp.finfo(jnp.float32).max
SCALE = DIM_HEAD ** -0.5

QB = 256
KB = 256
NKT = N // KB


def _proj_kernel(x_ref, g_ref, wq_ref, ws_ref, bs_ref, qkv_ref, comb_ref):
    x = x_ref[...]
    ms = jnp.mean(x * x, axis=1, keepdims=True)
    xn = x * jax.lax.rsqrt(ms + EPS) * g_ref[...]
    qkv_ref[...] = jnp.dot(xn, wq_ref[...], preferred_element_type=jnp.float32)
    comb_ref[...] = jax.nn.sigmoid(
        jnp.dot(xn, ws_ref[...], preferred_element_type=jnp.float32) + bs_ref[...])


def _comp_kernel(q_ref, k_ref, kb_ref, vb_ref, kw_ref, vw_ref, kcb_ref,
                 vcb_ref, kpos_ref, vpos_ref, memk_ref, memv_ref,
                 cos_ref, sin_ref, p_ref,
                 comp_ref, selm_ref, rq_ref, rk_ref):
    q = q_ref[0]
    ck = jnp.dot(kb_ref[0] + kpos_ref[0], kw_ref[0],
                 preferred_element_type=jnp.float32) + kcb_ref[0]
    cv = jnp.dot(vb_ref[0] + vpos_ref[0], vw_ref[0],
                 preferred_element_type=jnp.float32) + vcb_ref[0]

    dn = (((1,), (1,)), ((), ()))
    sim_mem = jax.lax.dot_general(q, memk_ref[0], dn,
                                  preferred_element_type=jnp.float32) * SCALE
    sim_ck = jax.lax.dot_general(q, ck, dn,
                                 preferred_element_type=jnp.float32) * SCALE

    ii = jax.lax.broadcasted_iota(jnp.int32, (N, NC), 0)
    cc = jax.lax.broadcasted_iota(jnp.int32, (N, NC), 1)
    vis = cc < (ii // BLOCK)
    sim_ck = jnp.where(vis, sim_ck, MASKVAL)

    m = jnp.maximum(jnp.max(sim_mem, axis=1, keepdims=True),
                    jnp.max(sim_ck, axis=1, keepdims=True))
    e_mem = jnp.exp(sim_mem - m)
    e_ck = jnp.exp(sim_ck - m)
    l = (jnp.sum(e_mem, axis=1, keepdims=True)
         + jnp.sum(e_ck, axis=1, keepdims=True))
    comp_ref[0] = (jnp.dot(e_mem, memv_ref[0], preferred_element_type=jnp.float32)
                   + jnp.dot(e_ck, cv, preferred_element_type=jnp.float32)) / l

    work = e_ck
    for _ in range(NUM_SEL - 1):
        mx = jnp.max(work, axis=1, keepdims=True)
        work = jnp.where(work == mx, -1.0, work)
    t4 = jnp.max(work, axis=1, keepdims=True)
    selm_ref[0] = jnp.where((e_ck >= t4) & (e_ck > 1e-10 * l), 1.0, 0.0)

    cos = cos_ref[...]
    sin = sin_ref[...]
    pm = p_ref[...]
    rq_ref[0] = (q * cos
                 + jnp.dot(q, pm, preferred_element_type=jnp.float32) * sin) * SCALE
    k = k_ref[0]
    rk_ref[0] = (k * cos
                 + jnp.dot(k, pm, preferred_element_type=jnp.float32) * sin)


def _attn_kernel(rq_ref, rk_ref, v_ref, e3_ref, selm_ref, comp_ref,
                 gates_ref, out_ref):
    qb = pl.program_id(1)
    rq = rq_ref[0]
    selm = selm_ref[0]

    ri = jax.lax.broadcasted_iota(jnp.int32, (QB, KB), 0)
    rj = jax.lax.broadcasted_iota(jnp.int32, (QB, KB), 1)
    gi = qb * QB + ri

    def tile(kb):
        rk = rk_ref[0, pl.ds(kb * KB, KB), :]
        vs = v_ref[0, pl.ds(kb * KB, KB), :]
        s = jax.lax.dot_general(rq, rk, (((1,), (1,)), ((), ())),
                                preferred_element_type=jnp.float32)
        fm16 = jnp.dot(selm, e3_ref[kb], preferred_element_type=jnp.float32)
        return s, vs, fm16 > 0.5

    def fupdate(carry, s, fm, vs):
        mf, lf, accf = carry
        sf = jnp.where(fm, s, MASKVAL)
        mf_n = jnp.maximum(mf, jnp.max(sf, axis=1, keepdims=True))
        pf = jnp.where(fm, jnp.exp(sf - mf_n), 0.0)
        cf = jnp.exp(mf - mf_n)
        lf = lf * cf + jnp.sum(pf, axis=1, keepdims=True)
        accf = accf * cf + jnp.dot(pf, vs, preferred_element_type=jnp.float32)
        return mf_n, lf, accf

    def supdate(carry, s, swm, vs):
        mw, lw, accw = carry
        sw = jnp.where(swm, s, MASKVAL)
        mw_n = jnp.maximum(mw, jnp.max(sw, axis=1, keepdims=True))
        pw = jnp.where(swm, jnp.exp(sw - mw_n), 0.0)
        cw = jnp.exp(mw - mw_n)
        lw = lw * cw + jnp.sum(pw, axis=1, keepdims=True)
        accw = accw * cw + jnp.dot(pw, vs, preferred_element_type=jnp.float32)
        return mw_n, lw, accw

    def body(kb, carry):
        s, vs, fm = tile(kb)
        return fupdate(carry, s, fm, vs)

    finit = (jnp.full((QB, 1), MASKVAL), jnp.zeros((QB, 1)),
             jnp.zeros((QB, DIM_HEAD)))
    fcarry = jax.lax.fori_loop(0, jnp.maximum(qb - 1, 0), body, finit)

    def tail(kb, diag, fcarry, scarry):
        s, vs, fm = tile(kb)
        if diag:
            gj = kb * KB + rj
            causal = gj <= gi
            fm = fm | (causal & ((gi // BLOCK) == (gj // BLOCK)))
            swm = causal & (gi - gj <= WINDOW)
        else:
            gj = kb * KB + rj
            swm = (gj <= gi) & (gi - gj <= WINDOW)
        fcarry = fupdate(fcarry, s, fm, vs)
        scarry = supdate(scarry, s, swm, vs)
        return fcarry, scarry

    sinit = (jnp.full((QB, 1), MASKVAL), jnp.zeros((QB, 1)),
             jnp.zeros((QB, DIM_HEAD)))
    fcarry, scarry = jax.lax.cond(
        qb > 0,
        lambda fc, sc: tail(qb - 1, False, fc, sc),
        lambda fc, sc: (fc, sc),
        fcarry, sinit)
    (mf, lf, accf), (mw, lw, accw) = tail(qb, True, fcarry, scarry)

    fine = accf / lf
    swout = accw / lw
    g = gates_ref[0]
    out_ref[0] = (g[:, 0:1] * comp_ref[0] + g[:, 1:2] * fine
                  + g[:, 2:3] * swout)


def _out_kernel(x_ref, w_ref, o_ref):
    o_ref[...] = jnp.dot(x_ref[...], w_ref[...],
                         preferred_element_type=jnp.float32)


def _rotary_tables():
    inv = 1.0 / (10000.0 ** (np.arange(0, DIM_HEAD, 2, dtype=np.float32)
                             / DIM_HEAD))
    freqs = np.arange(N, dtype=np.float32)[:, None] * inv[None, :]
    freqs = np.repeat(freqs, 2, axis=-1)
    cos = jnp.asarray(np.cos(freqs))
    sin = jnp.asarray(np.sin(freqs))
    p = np.zeros((DIM_HEAD, DIM_HEAD), np.float32)
    for j in range(0, DIM_HEAD, 2):
        p[j + 1, j] = -1.0
        p[j, j + 1] = 1.0
    return cos, sin, jnp.asarray(p)


def _expand_table():
    e3 = np.zeros((NKT, NC, KB), np.float32)
    for t in range(NKT):
        for jb in range(KB // BLOCK):
            e3[t, t * (KB // BLOCK) + jb, jb * BLOCK:(jb + 1) * BLOCK] = 1.0
    return jnp.asarray(e3)


def kernel(inp, norm_g, qkv_w, mem_kv, k_pos, v_pos, k_cw, k_cb, v_cw, v_cb,
           strat_w, strat_b, out_w):
    x2 = inp.reshape(N, DIM)
    gate_pad = 128
    ws = jnp.zeros((DIM, gate_pad), jnp.float32).at[:, :3 * HEADS].set(strat_w)
    bs = jnp.zeros((1, gate_pad), jnp.float32).at[0, :3 * HEADS].set(strat_b)

    qkv, comb = pl.pallas_call(
        _proj_kernel,
        grid=(N // QB,),
        in_specs=[
            pl.BlockSpec((QB, DIM), lambda i: (i, 0)),
            pl.BlockSpec((1, DIM), lambda i: (0, 0)),
            pl.BlockSpec((DIM, 3 * DIM_INNER), lambda i: (0, 0)),
            pl.BlockSpec((DIM, gate_pad), lambda i: (0, 0)),
            pl.BlockSpec((1, gate_pad), lambda i: (0, 0)),
        ],
        out_specs=[
            pl.BlockSpec((QB, 3 * DIM_INNER), lambda i: (i, 0)),
            pl.BlockSpec((QB, gate_pad), lambda i: (i, 0)),
        ],
        out_shape=[
            jax.ShapeDtypeStruct((N, 3 * DIM_INNER), jnp.float32),
            jax.ShapeDtypeStruct((N, gate_pad), jnp.float32),
        ],
    )(x2, norm_g.reshape(1, DIM), qkv_w, ws, bs)

    q = qkv[:, :DIM_INNER].reshape(N, HEADS, DIM_HEAD).transpose(1, 0, 2)
    k = qkv[:, DIM_INNER:2 * DIM_INNER].reshape(N, HEADS, DIM_HEAD).transpose(1, 0, 2)
    v = qkv[:, 2 * DIM_INNER:].reshape(N, HEADS, DIM_HEAD).transpose(1, 0, 2)
    gates = comb[:, :3 * HEADS].reshape(N, HEADS, 3).transpose(1, 0, 2)

    kblk = k.reshape(HEADS, NC, BLOCK * DIM_HEAD)
    vblk = v.reshape(HEADS, NC, BLOCK * DIM_HEAD)
    kw = k_cw.transpose(0, 3, 2, 1).reshape(HEADS, BLOCK * DIM_HEAD, DIM_HEAD)
    vw = v_cw.transpose(0, 3, 2, 1).reshape(HEADS, BLOCK * DIM_HEAD, DIM_HEAD)
    kpos_f = k_pos.reshape(HEADS, 1, BLOCK * DIM_HEAD)
    vpos_f = v_pos.reshape(HEADS, 1, BLOCK * DIM_HEAD)
    kcb2 = k_cb.reshape(HEADS, 1, DIM_HEAD)
    vcb2 = v_cb.reshape(HEADS, 1, DIM_HEAD)
    mem_k = mem_kv[0]
    mem_v = mem_kv[1]

    cos, sin, pmat = _rotary_tables()

    hfix = lambda shape: pl.BlockSpec((1,) + shape, lambda h: (h, 0, 0))
    hbc2 = lambda shape: pl.BlockSpec(shape, lambda h: (0, 0))
    comp, selm, rq, rk = pl.pallas_call(
        _comp_kernel,
        grid=(HEADS,),
        in_specs=[
            hfix((N, DIM_HEAD)),
            hfix((N, DIM_HEAD)),
            hfix((NC, BLOCK * DIM_HEAD)),
            hfix((NC, BLOCK * DIM_HEAD)),
            hfix((BLOCK * DIM_HEAD, DIM_HEAD)),
            hfix((BLOCK * DIM_HEAD, DIM_HEAD)),
            hfix((1, DIM_HEAD)),
            hfix((1, DIM_HEAD)),
            hfix((1, BLOCK * DIM_HEAD)),
            hfix((1, BLOCK * DIM_HEAD)),
            hfix((NUM_MEM, DIM_HEAD)),
            hfix((NUM_MEM, DIM_HEAD)),
            hbc2((N, DIM_HEAD)),
            hbc2((N, DIM_HEAD)),
            hbc2((DIM_HEAD, DIM_HEAD)),
        ],
        out_specs=[
            hfix((N, DIM_HEAD)),
            hfix((N, NC)),
            hfix((N, DIM_HEAD)),
            hfix((N, DIM_HEAD)),
        ],
        out_shape=[
            jax.ShapeDtypeStruct((HEADS, N, DIM_HEAD), jnp.float32),
            jax.ShapeDtypeStruct((HEADS, N, NC), jnp.float32),
            jax.ShapeDtypeStruct((HEADS, N, DIM_HEAD), jnp.float32),
            jax.ShapeDtypeStruct((HEADS, N, DIM_HEAD), jnp.float32),
        ],
    )(q, k, kblk, vblk, kw, vw, kcb2, vcb2, kpos_f, vpos_f, mem_k, mem_v,
      cos, sin, pmat)

    e3 = _expand_table()

    outc = pl.pallas_call(
        _attn_kernel,
        grid=(HEADS, N // QB),
        in_specs=[
            pl.BlockSpec((1, QB, DIM_HEAD), lambda h, i: (h, i, 0)),
            pl.BlockSpec((1, N, DIM_HEAD), lambda h, i: (h, 0, 0)),
            pl.BlockSpec((1, N, DIM_HEAD), lambda h, i: (h, 0, 0)),
            pl.BlockSpec((NKT, NC, KB), lambda h, i: (0, 0, 0)),
            pl.BlockSpec((1, QB, NC), lambda h, i: (h, i, 0)),
            pl.BlockSpec((1, QB, DIM_HEAD), lambda h, i: (h, i, 0)),
            pl.BlockSpec((1, QB, 3), lambda h, i: (h, i, 0)),
        ],
        out_specs=pl.BlockSpec((1, QB, DIM_HEAD), lambda h, i: (h, i, 0)),
        out_shape=jax.ShapeDtypeStruct((HEADS, N, DIM_HEAD), jnp.float32),
    )(rq, rk, v, e3, selm, comp, gates)

    merged = outc.transpose(1, 0, 2).reshape(N, DIM_INNER)

    out = pl.pallas_call(
        _out_kernel,
        grid=(N // QB,),
        in_specs=[
            pl.BlockSpec((QB, DIM_INNER), lambda i: (i, 0)),
            pl.BlockSpec((DIM_INNER, DIM), lambda i: (0, 0)),
        ],
        out_specs=pl.BlockSpec((QB, DIM), lambda i: (i, 0)),
        out_shape=jax.ShapeDtypeStruct((N, DIM), jnp.float32),
    )(merged, out_w)

    return out.reshape(B, N, DIM)
